# baseline (speedup 1.0000x reference)
import jax
import jax.numpy as jnp
from jax.experimental import pallas as pl
from jax.experimental.pallas import tpu as pltpu

_NUM_EXPERTS = 8
_TOPK = 2
_BLOCK = 2048


def _router_block(x_ref, wt_ref, b_ref, w_out, i_out, p_out):
    x = x_ref[...]
    logits = jax.lax.dot_general(
        x, wt_ref[...], (((1,), (0,)), ((), ())),
        preferred_element_type=jnp.float32,
    ) + b_ref[...]
    probs = jax.nn.sigmoid(logits)
    p_out[...] = probs

    iota = jax.lax.broadcasted_iota(jnp.int32, probs.shape, 1)
    m1 = jnp.max(probs, axis=1, keepdims=True)
    i1 = jnp.min(jnp.where(probs == m1, iota, _NUM_EXPERTS), axis=1,
                 keepdims=True)
    masked = jnp.where(iota == i1, -jnp.inf, probs)
    m2 = jnp.max(masked, axis=1, keepdims=True)
    i2 = jnp.min(jnp.where(masked == m2, iota, _NUM_EXPERTS), axis=1,
                 keepdims=True)
    s = m1 + m2
    w_out[...] = jnp.concatenate([m1 / s, m2 / s], axis=1)
    i_out[...] = jnp.concatenate([i1, i2], axis=1)


def kernel(hidden_states, W, b):
    n_tokens, hidden = hidden_states.shape
    n_exp = W.shape[0]
    wt = W.T
    b2 = b.reshape(1, n_exp)
    grid = (n_tokens // _BLOCK,)
    w_o, i_o, p_o = pl.pallas_call(
        _router_block,
        grid=grid,
        in_specs=[
            pl.BlockSpec((_BLOCK, hidden), lambda i: (i, 0)),
            pl.BlockSpec((hidden, n_exp), lambda i: (0, 0)),
            pl.BlockSpec((1, n_exp), lambda i: (0, 0)),
        ],
        out_specs=[
            pl.BlockSpec((_BLOCK, _TOPK), lambda i: (i, 0)),
            pl.BlockSpec((_BLOCK, _TOPK), lambda i: (i, 0)),
            pl.BlockSpec((_BLOCK, n_exp), lambda i: (i, 0)),
        ],
        out_shape=[
            jax.ShapeDtypeStruct((n_tokens, _TOPK), jnp.float32),
            jax.ShapeDtypeStruct((n_tokens, _TOPK), jnp.int32),
            jax.ShapeDtypeStruct((n_tokens, n_exp), jnp.float32),
        ],
    )(hidden_states, wt, b2)
    return (w_o, i_o, p_o)
